# baseline (speedup 1.0000x reference)
import jax
import jax.numpy as jnp
from jax.experimental import pallas as pl
from jax.experimental.pallas import tpu as pltpu

_BS = 8
_NSLOT = 3


def _composite_kernel(alpha_hbm, color_hbm, out_ref,
                      abuf, cbuf, taubuf, acc_ref, t_ref, k_ref,
                      sem_a, sem_c, *, ns, nb):
    b = pl.program_id(0)

    def start_alpha(bb, jb, dst_slot):
        src_j = (ns - 1) - jb
        pltpu.make_async_copy(
            alpha_hbm.at[bb, pl.ds(src_j * _BS, _BS)],
            abuf.at[dst_slot], sem_a.at[dst_slot]).start()

    def start_color(bb, jb, dst_slot):
        src_j = (ns - 1) - jb
        pltpu.make_async_copy(
            color_hbm.at[bb, pl.ds(src_j * _BS, _BS)],
            cbuf.at[dst_slot], sem_c.at[dst_slot]).start()

    def wait_alpha(dst_slot):
        pltpu.make_async_copy(
            alpha_hbm.at[0, pl.ds(0, _BS)],
            abuf.at[dst_slot], sem_a.at[dst_slot]).wait()

    def wait_color(dst_slot):
        pltpu.make_async_copy(
            color_hbm.at[0, pl.ds(0, _BS)],
            cbuf.at[dst_slot], sem_c.at[dst_slot]).wait()

    acc_ref[...] = jnp.zeros_like(acc_ref)
    t_ref[...] = jnp.ones_like(t_ref)
    k_ref[...] = jnp.zeros_like(k_ref)

    @pl.when(b == 0)
    def _prologue():
        start_alpha(0, 0, 0)
        if ns > 1:
            start_alpha(0, 1, 1)
        start_color(0, 0, 0)

    def cond(carry):
        jb, done, _ = carry
        return jnp.logical_and(jb < ns, done == 0)

    def body(carry):
        jb, done, colp = carry
        slot = jax.lax.rem(jb, _NSLOT)

        @pl.when(jnp.logical_and(jb + 1 < ns, done == 0))
        def _fetch_color():
            start_color(b, jb + 1, jax.lax.rem(jb + 1, _NSLOT))
        colp_next = jnp.where(jb + 1 < ns, 1 - done, 0)

        wait_alpha(slot)

        @pl.when(jb + 2 < ns)
        def _prefetch_alpha():
            start_alpha(b, jb + 2, jax.lax.rem(jb + 2, _NSLOT))

        t = t_ref[...]
        k = k_ref[...]
        for s in reversed(range(_BS)):
            a = abuf[slot, s]
            sel = jnp.logical_and(a > 0.1, k < 10)
            ta = t * jnp.where(sel, a, 0.0)
            taubuf[slot, s] = ta
            t = t - ta
            k = k + sel.astype(jnp.int32)
        t_ref[...] = t
        k_ref[...] = k
        done_now = (jnp.min(k) >= 10).astype(jnp.int32)

        @pl.when(colp == 1)
        def _apply_color():
            wait_color(slot)
            acc = acc_ref[...]
            for s in range(_BS):
                acc = acc + taubuf[slot, s][None, :, :] * cbuf[slot, s]
            acc_ref[...] = acc

        return jb + 1, done_now, colp_next

    jb_exit, done_final, colp_final = jax.lax.while_loop(cond, body, (0, 0, 1))

    @pl.when(colp_final == 1)
    def _drainc():
        wait_color(jax.lax.rem(jb_exit, _NSLOT))

    @pl.when(jb_exit < ns)
    def _drain0():
        wait_alpha(jax.lax.rem(jb_exit, _NSLOT))

    @pl.when(jb_exit + 1 < ns)
    def _drain1():
        wait_alpha(jax.lax.rem(jb_exit + 1, _NSLOT))

    @pl.when(done_final == 0)
    def _pass2():
        acc = acc_ref[...]
        t = t_ref[...]
        k = k_ref[...]
        for p in range(_NSLOT):
            pslot = (ns - 1 - p) % _NSLOT
            for s in range(_BS):
                a = abuf[pslot, s]
                sel = jnp.logical_and(a <= 0.1, k < 10)
                ta = t * jnp.where(sel, a, 0.0)
                acc = acc + ta[None, :, :] * cbuf[pslot, s]
                t = t - ta
                k = k + sel.astype(jnp.int32)
        acc_ref[...] = acc
        t_ref[...] = t

    out_ref[0] = acc_ref[...] + t_ref[...][None, :, :]

    @pl.when(b + 1 < nb)
    def _next_image():
        start_alpha(b + 1, 0, 0)
        if ns > 1:
            start_alpha(b + 1, 1, 1)
        start_color(b + 1, 0, 0)


def kernel(color_stroke, alpha):
    b, s, _, w, _ = color_stroke.shape
    ns = s // _BS
    alpha2 = alpha.reshape(b, s, w, w)

    out = pl.pallas_call(
        lambda ar, cr, orf, *rest: _composite_kernel(
            ar, cr, orf, *rest, ns=ns, nb=b),
        grid=(b,),
        in_specs=[
            pl.BlockSpec(memory_space=pl.ANY),
            pl.BlockSpec(memory_space=pl.ANY),
        ],
        out_specs=pl.BlockSpec((1, 3, w, w), lambda i: (i, 0, 0, 0)),
        out_shape=jax.ShapeDtypeStruct((b, 3, w, w), color_stroke.dtype),
        scratch_shapes=[
            pltpu.VMEM((_NSLOT, _BS, w, w), jnp.float32),
            pltpu.VMEM((_NSLOT, _BS, 3, w, w), jnp.float32),
            pltpu.VMEM((_NSLOT, _BS, w, w), jnp.float32),
            pltpu.VMEM((3, w, w), jnp.float32),
            pltpu.VMEM((w, w), jnp.float32),
            pltpu.VMEM((w, w), jnp.int32),
            pltpu.SemaphoreType.DMA((_NSLOT,)),
            pltpu.SemaphoreType.DMA((_NSLOT,)),
        ],
        compiler_params=pltpu.CompilerParams(
            dimension_semantics=("arbitrary",)),
    )(alpha2, color_stroke)
    return out


# speedup vs baseline: 1.0126x; 1.0126x over previous
import jax
import jax.numpy as jnp
from jax.experimental import pallas as pl
from jax.experimental.pallas import tpu as pltpu

_BS = 8
_NSLOT = 3


def _composite_kernel(alpha_hbm, color_hbm, out_ref,
                      abuf, cbuf, taubuf, acc_ref, t_ref, k_ref,
                      sem_a, sem_c, *, ns, nb):
    b = pl.program_id(0)

    def start_alpha(bb, jb, dst_slot):
        src_j = (ns - 1) - jb
        pltpu.make_async_copy(
            alpha_hbm.at[bb, pl.ds(src_j * _BS, _BS)],
            abuf.at[dst_slot], sem_a.at[dst_slot]).start()

    def start_color(bb, jb, dst_slot):
        src_j = (ns - 1) - jb
        pltpu.make_async_copy(
            color_hbm.at[bb, pl.ds(src_j * _BS, _BS)],
            cbuf.at[dst_slot], sem_c.at[dst_slot]).start()

    def wait_alpha(dst_slot):
        pltpu.make_async_copy(
            alpha_hbm.at[0, pl.ds(0, _BS)],
            abuf.at[dst_slot], sem_a.at[dst_slot]).wait()

    def wait_color(dst_slot):
        pltpu.make_async_copy(
            color_hbm.at[0, pl.ds(0, _BS)],
            cbuf.at[dst_slot], sem_c.at[dst_slot]).wait()

    acc_ref[...] = jnp.zeros_like(acc_ref)
    t_ref[...] = jnp.ones_like(t_ref)
    k_ref[...] = jnp.zeros_like(k_ref)

    @pl.when(b == 0)
    def _prologue():
        start_alpha(0, 0, 0)
        if ns > 1:
            start_alpha(0, 1, 1)
        start_color(0, 0, 0)

    def cond(carry):
        jb, done, _ = carry
        return jnp.logical_and(jb < ns, done == 0)

    def body(carry):
        jb, done, colp = carry
        slot = jax.lax.rem(jb, _NSLOT)
        wait_alpha(slot)

        @pl.when(jb + 2 < ns)
        def _prefetch_alpha():
            start_alpha(b, jb + 2, jax.lax.rem(jb + 2, _NSLOT))

        t = t_ref[...]
        k = k_ref[...]
        for s in reversed(range(_BS)):
            a = abuf[slot, s]
            sel = jnp.logical_and(a > 0.1, k < 10)
            ta = t * jnp.where(sel, a, 0.0)
            taubuf[slot, s] = ta
            t = t - ta
            k = k + sel.astype(jnp.int32)
        t_ref[...] = t
        k_ref[...] = k
        done_now = (jnp.min(k) >= 10).astype(jnp.int32)

        @pl.when(jnp.logical_and(jb + 1 < ns, done_now == 0))
        def _fetch_color():
            start_color(b, jb + 1, jax.lax.rem(jb + 1, _NSLOT))

        @pl.when(colp == 1)
        def _apply_color():
            wait_color(slot)
            acc = acc_ref[...]
            for s in range(_BS):
                acc = acc + taubuf[slot, s][None, :, :] * cbuf[slot, s]
            acc_ref[...] = acc

        new_colp = jnp.where(jb + 1 < ns, 1 - done_now, 0)
        return jb + 1, done_now, new_colp

    jb_exit, done_final, _ = jax.lax.while_loop(cond, body, (0, 0, 1))

    @pl.when(jb_exit < ns)
    def _drain0():
        wait_alpha(jax.lax.rem(jb_exit, _NSLOT))

    @pl.when(jb_exit + 1 < ns)
    def _drain1():
        wait_alpha(jax.lax.rem(jb_exit + 1, _NSLOT))

    @pl.when(done_final == 0)
    def _pass2():
        acc = acc_ref[...]
        t = t_ref[...]
        k = k_ref[...]
        for p in range(_NSLOT):
            pslot = (ns - 1 - p) % _NSLOT
            for s in range(_BS):
                a = abuf[pslot, s]
                sel = jnp.logical_and(a <= 0.1, k < 10)
                ta = t * jnp.where(sel, a, 0.0)
                acc = acc + ta[None, :, :] * cbuf[pslot, s]
                t = t - ta
                k = k + sel.astype(jnp.int32)
        acc_ref[...] = acc
        t_ref[...] = t

    out_ref[0] = acc_ref[...] + t_ref[...][None, :, :]

    @pl.when(b + 1 < nb)
    def _next_image():
        start_alpha(b + 1, 0, 0)
        if ns > 1:
            start_alpha(b + 1, 1, 1)
        start_color(b + 1, 0, 0)


def kernel(color_stroke, alpha):
    b, s, _, w, _ = color_stroke.shape
    ns = s // _BS
    alpha2 = alpha.reshape(b, s, w, w)

    out = pl.pallas_call(
        lambda ar, cr, orf, *rest: _composite_kernel(
            ar, cr, orf, *rest, ns=ns, nb=b),
        grid=(b,),
        in_specs=[
            pl.BlockSpec(memory_space=pl.ANY),
            pl.BlockSpec(memory_space=pl.ANY),
        ],
        out_specs=pl.BlockSpec((1, 3, w, w), lambda i: (i, 0, 0, 0)),
        out_shape=jax.ShapeDtypeStruct((b, 3, w, w), color_stroke.dtype),
        scratch_shapes=[
            pltpu.VMEM((_NSLOT, _BS, w, w), jnp.float32),
            pltpu.VMEM((_NSLOT, _BS, 3, w, w), jnp.float32),
            pltpu.VMEM((_NSLOT, _BS, w, w), jnp.float32),
            pltpu.VMEM((3, w, w), jnp.float32),
            pltpu.VMEM((w, w), jnp.float32),
            pltpu.VMEM((w, w), jnp.int32),
            pltpu.SemaphoreType.DMA((_NSLOT,)),
            pltpu.SemaphoreType.DMA((_NSLOT,)),
        ],
        compiler_params=pltpu.CompilerParams(
            dimension_semantics=("arbitrary",)),
    )(alpha2, color_stroke)
    return out


# speedup vs baseline: 1.3801x; 1.3628x over previous
import jax
import jax.numpy as jnp
from jax.experimental import pallas as pl
from jax.experimental.pallas import tpu as pltpu

_BS = 8
_NSLOT = 3


def _composite_kernel(alpha_hbm, color_hbm, out_ref,
                      abuf, cbuf, taubuf, acc_ref, t_ref, k_ref,
                      sem_a, sem_c, *, nb, ns):
    def start_alpha(bb, jb, dst_slot):
        src_j = (ns - 1) - jb
        pltpu.make_async_copy(
            alpha_hbm.at[bb, pl.ds(src_j * _BS, _BS)],
            abuf.at[bb, dst_slot], sem_a.at[bb, dst_slot]).start()

    def start_color(bb, jb, dst_slot):
        src_j = (ns - 1) - jb
        pltpu.make_async_copy(
            color_hbm.at[bb, pl.ds(src_j * _BS, _BS)],
            cbuf.at[bb, dst_slot], sem_c.at[bb, dst_slot]).start()

    def wait_alpha(bb, dst_slot):
        pltpu.make_async_copy(
            alpha_hbm.at[0, pl.ds(0, _BS)],
            abuf.at[bb, dst_slot], sem_a.at[bb, dst_slot]).wait()

    def wait_color(bb, dst_slot):
        pltpu.make_async_copy(
            color_hbm.at[0, pl.ds(0, _BS)],
            cbuf.at[bb, dst_slot], sem_c.at[bb, dst_slot]).wait()

    acc_ref[...] = jnp.zeros_like(acc_ref)
    t_ref[...] = jnp.ones_like(t_ref)
    k_ref[...] = jnp.zeros_like(k_ref)

    for bb in range(nb):
        start_alpha(bb, 0, 0)
        if ns > 1:
            start_alpha(bb, 1, 1)
        start_color(bb, 0, 0)

    def cond(carry):
        jb = carry[0]
        act = carry[1]
        return jnp.logical_and(jb < ns, sum(act) > 0)

    def body(carry):
        jb, act, msat = carry
        slot = jax.lax.rem(jb, _NSLOT)

        for bb in range(nb):
            @pl.when(act[bb] == 1)
            def _alpha(bb=bb):
                wait_alpha(bb, slot)

                @pl.when(jb + 2 < ns)
                def _prefetch():
                    start_alpha(bb, jb + 2, jax.lax.rem(jb + 2, _NSLOT))

                t = t_ref[bb]
                k = k_ref[bb]
                for s in reversed(range(_BS)):
                    a = abuf[bb, slot, s]
                    sel = jnp.logical_and(a > 0.1, k < 10)
                    ta = t * jnp.where(sel, a, 0.0)
                    taubuf[bb, slot, s] = ta
                    t = t - ta
                    k = k + sel.astype(jnp.int32)
                t_ref[bb] = t
                k_ref[bb] = k

        done_new = [(jnp.min(k_ref[bb]) >= 10).astype(jnp.int32)
                    for bb in range(nb)]

        for bb in range(nb):
            @pl.when(jnp.logical_and(
                act[bb] == 1,
                jnp.logical_and(jb + 1 < ns, done_new[bb] == 0)))
            def _fetch_color(bb=bb):
                start_color(bb, jb + 1, jax.lax.rem(jb + 1, _NSLOT))

        for bb in range(nb):
            @pl.when(act[bb] == 1)
            def _color(bb=bb):
                wait_color(bb, slot)
                acc = acc_ref[bb]
                for s in range(_BS):
                    acc = acc + taubuf[bb, slot, s][None] * cbuf[bb, slot, s]
                acc_ref[bb] = acc

        act_new = tuple(
            jnp.where(act[bb] == 1, 1 - done_new[bb], 0) for bb in range(nb))
        msat_new = tuple(
            jnp.where(jnp.logical_and(act[bb] == 1, done_new[bb] == 1),
                      jb, msat[bb]) for bb in range(nb))
        return jb + 1, act_new, msat_new

    _, act_fin, msat_fin = jax.lax.while_loop(
        cond, body, (0, (1,) * nb, (ns,) * nb))

    for bb in range(nb):
        @pl.when(msat_fin[bb] + 1 < ns)
        def _drain0(bb=bb):
            wait_alpha(bb, jax.lax.rem(msat_fin[bb] + 1, _NSLOT))

        @pl.when(msat_fin[bb] + 2 < ns)
        def _drain1(bb=bb):
            wait_alpha(bb, jax.lax.rem(msat_fin[bb] + 2, _NSLOT))

        @pl.when(act_fin[bb] == 1)
        def _pass2(bb=bb):
            acc = acc_ref[bb]
            t = t_ref[bb]
            k = k_ref[bb]
            for p in range(_NSLOT):
                pslot = (ns - 1 - p) % _NSLOT
                for s in range(_BS):
                    a = abuf[bb, pslot, s]
                    sel = jnp.logical_and(a <= 0.1, k < 10)
                    ta = t * jnp.where(sel, a, 0.0)
                    acc = acc + ta[None] * cbuf[bb, pslot, s]
                    t = t - ta
                    k = k + sel.astype(jnp.int32)
            acc_ref[bb] = acc
            t_ref[bb] = t

        out_ref[bb] = acc_ref[bb] + t_ref[bb][None]


def kernel(color_stroke, alpha):
    b, s, _, w, _ = color_stroke.shape
    ns = s // _BS
    alpha2 = alpha.reshape(b, s, w, w)

    out = pl.pallas_call(
        lambda ar, cr, orf, *rest: _composite_kernel(
            ar, cr, orf, *rest, nb=b, ns=ns),
        grid=(1,),
        in_specs=[
            pl.BlockSpec(memory_space=pl.ANY),
            pl.BlockSpec(memory_space=pl.ANY),
        ],
        out_specs=pl.BlockSpec((b, 3, w, w), lambda i: (0, 0, 0, 0)),
        out_shape=jax.ShapeDtypeStruct((b, 3, w, w), color_stroke.dtype),
        scratch_shapes=[
            pltpu.VMEM((b, _NSLOT, _BS, w, w), jnp.float32),
            pltpu.VMEM((b, _NSLOT, _BS, 3, w, w), jnp.float32),
            pltpu.VMEM((b, _NSLOT, _BS, w, w), jnp.float32),
            pltpu.VMEM((b, 3, w, w), jnp.float32),
            pltpu.VMEM((b, w, w), jnp.float32),
            pltpu.VMEM((b, w, w), jnp.int32),
            pltpu.SemaphoreType.DMA((b, _NSLOT)),
            pltpu.SemaphoreType.DMA((b, _NSLOT)),
        ],
        compiler_params=pltpu.CompilerParams(
            dimension_semantics=("arbitrary",)),
    )(alpha2, color_stroke)
    return out
